# baseline (speedup 1.0000x reference)
import functools

import jax
import jax.numpy as jnp
from jax import lax
from jax.experimental import pallas as pl
from jax.experimental.pallas import tpu as pltpu
from jax.experimental.pallas import tpu_sc as plsc

_N = 10000
_E = 320000
_G = 64
_F = 128
_C = 40
_CP = 64

_NC, _NS = 2, 16
_NW = _NC * _NS
_EPW = 10240
_EPAD = _NW * _EPW
_NPAD = 10240
_ZCH = 16
_KPT = _NPAD // _NS // _ZCH


def _make_edge_agg(d, ch, sch, nbuf, tc_tiling=True):
  nchunk = _EPW // ch
  nslab = nchunk // sch
  m = sch // nbuf
  mesh = plsc.VectorSubcoreMesh(core_axis_name="c", subcore_axis_name="s")
  extra = {}
  if not tc_tiling:
    extra["compiler_params"] = pltpu.CompilerParams(use_tc_tiling_on_sc=False)

  @functools.partial(
      pl.kernel,
      mesh=mesh,
      **extra,
      out_type=jax.ShapeDtypeStruct((_NC, _NPAD, d), jnp.float32),
      scratch_types=[
          pltpu.VMEM((2, sch, ch), jnp.int32),
          pltpu.VMEM((nbuf, ch, d), jnp.float32),
          pltpu.VMEM((_ZCH, d), jnp.float32),
          pltpu.VMEM_SHARED((_NPAD, d), jnp.float32),
      ] + [pltpu.SemaphoreType.DMA] * (2 * nbuf),
  )
  def agg(table_hbm, src_hbm, dst_hbm, out_hbm,
          idx_sl, msgs_v, zbuf_v, acc_sh, *sems):
    gsem = sems[:nbuf]
    ssem = sems[nbuf:]
    c = lax.axis_index("c")
    s = lax.axis_index("s")
    wid = s * _NC + c

    def _zrow(i, carry):
      for j in range(d // 16):
        zbuf_v[i, pl.ds(j * 16, 16)] = jnp.zeros((16,), jnp.float32)
      return carry

    lax.fori_loop(0, _ZCH, _zrow, 0)
    for k in range(_KPT):
      pltpu.sync_copy(zbuf_v, acc_sh.at[pl.ds((s * _KPT + k) * _ZCH, _ZCH)])
    plsc.subcore_barrier()

    def _gather(j, b, sem):
      pltpu.async_copy(table_hbm.at[idx_sl.at[0, j]], msgs_v.at[b], sem)

    def _scatter(j, b, sem):
      pltpu.async_copy(msgs_v.at[b], acc_sh.at[idx_sl.at[1, j]], sem,
                       add=True)

    def _wait(sem, b):
      pltpu.make_async_copy(
          table_hbm.at[idx_sl.at[0, 0]], msgs_v.at[b], sem).wait()

    for sl in range(nslab):
      pltpu.sync_copy(src_hbm.at[wid, pl.ds(sl * sch, sch)], idx_sl.at[0])
      pltpu.sync_copy(dst_hbm.at[wid, pl.ds(sl * sch, sch)], idx_sl.at[1])
      for b in range(nbuf - 1):
        _gather(b, b, gsem[b])

      def _group(i, carry):
        for u in range(nbuf):
          _wait(gsem[u], u)
          _scatter(nbuf * i + u, u, ssem[u])
          if u == 0:
            @pl.when(i > 0)
            def _():
              _wait(ssem[nbuf - 1], nbuf - 1)
            _gather(nbuf * i + nbuf - 1, nbuf - 1, gsem[nbuf - 1])
          else:
            _wait(ssem[u - 1], u - 1)

            @pl.when(i < m - 1)
            def _():
              _gather(nbuf * i + u + nbuf - 1, u - 1, gsem[u - 1])
        return carry

      lax.fori_loop(0, m, _group, 0)
      _wait(ssem[nbuf - 1], nbuf - 1)
    plsc.subcore_barrier()

    for k in range(_KPT):
      r0 = (s * _KPT + k) * _ZCH
      pltpu.sync_copy(acc_sh.at[pl.ds(r0, _ZCH)], zbuf_v)
      pltpu.sync_copy(zbuf_v, out_hbm.at[c, pl.ds(r0, _ZCH)])

  return agg


_agg_x = _make_edge_agg(_F, 64, 40, 4)
_agg_q = _make_edge_agg(_CP, 128, 16, 8, tc_tiling=False)


def _tc_transform(p, w1, w2p):
  def body(p0, p1, a, b, o):
    t = jnp.dot(p0[...] + p1[...], a[...], preferred_element_type=jnp.float32)
    t = jnp.maximum(t, 0.0)
    o[...] = jnp.dot(t, b[...], preferred_element_type=jnp.float32)

  blk = 1280
  return pl.pallas_call(
      body,
      grid=(_NPAD // blk,),
      in_specs=[
          pl.BlockSpec((None, blk, _F), lambda i: (0, i, 0)),
          pl.BlockSpec((None, blk, _F), lambda i: (1, i, 0)),
          pl.BlockSpec((_F, _F), lambda i: (0, 0)),
          pl.BlockSpec((_F, _CP), lambda i: (0, 0)),
      ],
      out_specs=pl.BlockSpec((blk, _CP), lambda i: (i, 0)),
      out_shape=jax.ShapeDtypeStruct((_NPAD, _CP), jnp.float32),
  )(p, p, w1, w2p)


def _tc_pool(p2, batch2d):
  def body(p0, p1, b, o):
    h2 = p0[...] + p1[...]
    gids = lax.broadcasted_iota(jnp.int32, (_NPAD, _G), 1)
    onehot = jnp.where(b[...] == gids, 1.0, 0.0).astype(jnp.float32)
    sums = lax.dot_general(onehot, h2, (((0,), (0,)), ((), ())),
                           preferred_element_type=jnp.float32)
    counts = jnp.maximum(jnp.sum(onehot, axis=0), 1.0)
    o[...] = sums / counts[:, None]

  return pl.pallas_call(
      body,
      grid=(1,),
      in_specs=[
          pl.BlockSpec((None, _NPAD, _CP), lambda i: (0, 0, 0)),
          pl.BlockSpec((None, _NPAD, _CP), lambda i: (1, 0, 0)),
          pl.BlockSpec((_NPAD, 1), lambda i: (0, 0)),
      ],
      out_specs=pl.BlockSpec((_G, _CP), lambda i: (0, 0)),
      out_shape=jax.ShapeDtypeStruct((_G, _CP), jnp.float32),
  )(p2, p2, batch2d)


def kernel(x, edge_index, batch, W1, W2):
  src = edge_index[0].astype(jnp.int32)
  dst = edge_index[1].astype(jnp.int32)
  pad = _EPAD - _E
  ar = jnp.arange(pad, dtype=jnp.int32)
  src_p = jnp.concatenate([src, ar % _N])
  dst_p = jnp.concatenate([dst, _N + (ar % (_NPAD - _N))])

  p1 = _agg_x(x, src_p.reshape(_NW, _EPW // 64, 64),
              dst_p.reshape(_NW, _EPW // 64, 64))
  w2p = jnp.pad(W2, ((0, 0), (0, _CP - _C)))
  q = _tc_transform(p1, W1, w2p)
  p2 = _agg_q(q, src_p.reshape(_NW, _EPW // 128, 128),
              dst_p.reshape(_NW, _EPW // 128, 128))

  bpad = jnp.concatenate(
      [batch.astype(jnp.int32), jnp.full((_NPAD - _N,), _G, jnp.int32)])
  out = _tc_pool(p2, bpad.reshape(_NPAD, 1))
  return out[:, :_C]


# speedup vs baseline: 1.0427x; 1.0427x over previous
import functools

import jax
import jax.numpy as jnp
from jax import lax
from jax.experimental import pallas as pl
from jax.experimental.pallas import tpu as pltpu
from jax.experimental.pallas import tpu_sc as plsc

_N = 10000
_E = 320000
_G = 64
_F = 128
_C = 40
_CP = 64

_NC, _NS = 2, 16
_NW = _NC * _NS
_EPW = 10240
_EPAD = _NW * _EPW
_NPAD = 10240
_ZCH = 32
_KPT = _NPAD // _NS // _ZCH


def _make_edge_agg(d, ch, sch, nbuf, tc_tiling=True):
  nchunk = _EPW // ch
  nslab = nchunk // sch
  m = sch // nbuf
  mesh = plsc.VectorSubcoreMesh(core_axis_name="c", subcore_axis_name="s")
  extra = {}
  if not tc_tiling:
    extra["compiler_params"] = pltpu.CompilerParams(use_tc_tiling_on_sc=False)

  @functools.partial(
      pl.kernel,
      mesh=mesh,
      **extra,
      out_type=jax.ShapeDtypeStruct((_NC, _NPAD, d), jnp.float32),
      scratch_types=[
          pltpu.VMEM((2, sch, ch), jnp.int32),
          pltpu.VMEM((nbuf, ch, d), jnp.float32),
          pltpu.VMEM((_ZCH, d), jnp.float32),
          pltpu.VMEM_SHARED((_NPAD, d), jnp.float32),
      ] + [pltpu.SemaphoreType.DMA] * (2 * nbuf),
  )
  def agg(table_hbm, src_hbm, dst_hbm, out_hbm,
          idx_sl, msgs_v, zbuf_v, acc_sh, *sems):
    gsem = sems[:nbuf]
    ssem = sems[nbuf:]
    c = lax.axis_index("c")
    s = lax.axis_index("s")
    wid = s * _NC + c

    def _zrow(i, carry):
      for j in range(d // 16):
        zbuf_v[i, pl.ds(j * 16, 16)] = jnp.zeros((16,), jnp.float32)
      return carry

    lax.fori_loop(0, _ZCH, _zrow, 0)
    for k in range(_KPT):
      pltpu.sync_copy(zbuf_v, acc_sh.at[pl.ds((s * _KPT + k) * _ZCH, _ZCH)])
    plsc.subcore_barrier()

    def _gather(j, b, sem):
      pltpu.async_copy(table_hbm.at[idx_sl.at[0, j]], msgs_v.at[b], sem)

    def _scatter(j, b, sem):
      pltpu.async_copy(msgs_v.at[b], acc_sh.at[idx_sl.at[1, j]], sem,
                       add=True)

    def _wait(sem, b):
      pltpu.make_async_copy(
          table_hbm.at[idx_sl.at[0, 0]], msgs_v.at[b], sem).wait()

    for sl in range(nslab):
      pltpu.sync_copy(src_hbm.at[wid, pl.ds(sl * sch, sch)], idx_sl.at[0])
      pltpu.sync_copy(dst_hbm.at[wid, pl.ds(sl * sch, sch)], idx_sl.at[1])
      for b in range(nbuf - 1):
        _gather(b, b, gsem[b])

      def _group(i, carry):
        for u in range(nbuf):
          _wait(gsem[u], u)
          _scatter(nbuf * i + u, u, ssem[u])
          if u == 0:
            @pl.when(i > 0)
            def _():
              _wait(ssem[nbuf - 1], nbuf - 1)
            _gather(nbuf * i + nbuf - 1, nbuf - 1, gsem[nbuf - 1])
          else:
            _wait(ssem[u - 1], u - 1)

            @pl.when(i < m - 1)
            def _():
              _gather(nbuf * i + u + nbuf - 1, u - 1, gsem[u - 1])
        return carry

      lax.fori_loop(0, m, _group, 0)
      _wait(ssem[nbuf - 1], nbuf - 1)
    plsc.subcore_barrier()

    for k in range(_KPT):
      r0 = (s * _KPT + k) * _ZCH
      pltpu.sync_copy(acc_sh.at[pl.ds(r0, _ZCH)], zbuf_v)
      pltpu.sync_copy(zbuf_v, out_hbm.at[c, pl.ds(r0, _ZCH)])

  return agg


_agg_x = _make_edge_agg(_F, 64, 40, 4)
_agg_q = _make_edge_agg(_CP, 128, 16, 4, tc_tiling=False)


def _tc_transform(p, w1, w2p):
  def body(p0, p1, a, b, o):
    t = jnp.dot(p0[...] + p1[...], a[...], preferred_element_type=jnp.float32)
    t = jnp.maximum(t, 0.0)
    o[...] = jnp.dot(t, b[...], preferred_element_type=jnp.float32)

  blk = 1280
  return pl.pallas_call(
      body,
      grid=(_NPAD // blk,),
      in_specs=[
          pl.BlockSpec((None, blk, _F), lambda i: (0, i, 0)),
          pl.BlockSpec((None, blk, _F), lambda i: (1, i, 0)),
          pl.BlockSpec((_F, _F), lambda i: (0, 0)),
          pl.BlockSpec((_F, _CP), lambda i: (0, 0)),
      ],
      out_specs=pl.BlockSpec((blk, _CP), lambda i: (i, 0)),
      out_shape=jax.ShapeDtypeStruct((_NPAD, _CP), jnp.float32),
  )(p, p, w1, w2p)


def _tc_pool(p2, batch2d):
  def body(p0, p1, b, o):
    h2 = p0[...] + p1[...]
    gids = lax.broadcasted_iota(jnp.int32, (_NPAD, _G), 1)
    onehot = jnp.where(b[...] == gids, 1.0, 0.0).astype(jnp.float32)
    sums = lax.dot_general(onehot, h2, (((0,), (0,)), ((), ())),
                           preferred_element_type=jnp.float32)
    counts = jnp.maximum(jnp.sum(onehot, axis=0), 1.0)
    o[...] = sums / counts[:, None]

  return pl.pallas_call(
      body,
      grid=(1,),
      in_specs=[
          pl.BlockSpec((None, _NPAD, _CP), lambda i: (0, 0, 0)),
          pl.BlockSpec((None, _NPAD, _CP), lambda i: (1, 0, 0)),
          pl.BlockSpec((_NPAD, 1), lambda i: (0, 0)),
      ],
      out_specs=pl.BlockSpec((_G, _CP), lambda i: (0, 0)),
      out_shape=jax.ShapeDtypeStruct((_G, _CP), jnp.float32),
  )(p2, p2, batch2d)


def kernel(x, edge_index, batch, W1, W2):
  src = edge_index[0].astype(jnp.int32)
  dst = edge_index[1].astype(jnp.int32)
  pad = _EPAD - _E
  ar = jnp.arange(pad, dtype=jnp.int32)
  src_p = jnp.concatenate([src, ar % _N])
  dst_p = jnp.concatenate([dst, _N + (ar % (_NPAD - _N))])

  p1 = _agg_x(x, src_p.reshape(_NW, _EPW // 64, 64),
              dst_p.reshape(_NW, _EPW // 64, 64))
  w2p = jnp.pad(W2, ((0, 0), (0, _CP - _C)))
  q = _tc_transform(p1, W1, w2p)
  p2 = _agg_q(q, src_p.reshape(_NW, _EPW // 128, 128),
              dst_p.reshape(_NW, _EPW // 128, 128))

  bpad = jnp.concatenate(
      [batch.astype(jnp.int32), jnp.full((_NPAD - _N,), _G, jnp.int32)])
  out = _tc_pool(p2, bpad.reshape(_NPAD, 1))
  return out[:, :_C]


# speedup vs baseline: 1.0764x; 1.0324x over previous
import functools

import jax
import jax.numpy as jnp
from jax import lax
from jax.experimental import pallas as pl
from jax.experimental.pallas import tpu as pltpu
from jax.experimental.pallas import tpu_sc as plsc

_N = 10000
_E = 320000
_G = 64
_F = 128
_C = 40
_CP = 64

_NC, _NS = 2, 16
_NW = _NC * _NS
_EPW = 10240
_EPAD = _NW * _EPW
_NPAD = 10240
_ZCH = 32
_KPT = _NPAD // _NS // _ZCH


def _make_edge_agg(d, ch, sch, nbuf, tc_tiling=True):
  nchunk = _EPW // ch
  nslab = nchunk // sch
  m = sch // nbuf
  mesh = plsc.VectorSubcoreMesh(core_axis_name="c", subcore_axis_name="s")
  extra = {}
  if not tc_tiling:
    extra["compiler_params"] = pltpu.CompilerParams(use_tc_tiling_on_sc=False)

  @functools.partial(
      pl.kernel,
      mesh=mesh,
      **extra,
      out_type=jax.ShapeDtypeStruct((_NC, _NPAD, d), jnp.float32),
      scratch_types=[
          pltpu.VMEM((2, sch, ch), jnp.int32),
          pltpu.VMEM((nbuf, ch, d), jnp.float32),
          pltpu.VMEM((_ZCH, d), jnp.float32),
          pltpu.VMEM_SHARED((_NPAD, d), jnp.float32),
      ] + [pltpu.SemaphoreType.DMA] * (2 * nbuf),
  )
  def agg(table_hbm, src_hbm, dst_hbm, out_hbm,
          idx_sl, msgs_v, zbuf_v, acc_sh, *sems):
    gsem = sems[:nbuf]
    ssem = sems[nbuf:]
    c = lax.axis_index("c")
    s = lax.axis_index("s")
    wid = s * _NC + c

    def _zrow(i, carry):
      for j in range(d // 16):
        zbuf_v[i, pl.ds(j * 16, 16)] = jnp.zeros((16,), jnp.float32)
      return carry

    lax.fori_loop(0, _ZCH, _zrow, 0)
    for k in range(_KPT):
      pltpu.sync_copy(zbuf_v, acc_sh.at[pl.ds((s * _KPT + k) * _ZCH, _ZCH)])
    plsc.subcore_barrier()

    def _gather(j, b, sem):
      pltpu.async_copy(table_hbm.at[idx_sl.at[0, j]], msgs_v.at[b], sem)

    def _scatter(j, b, sem):
      pltpu.async_copy(msgs_v.at[b], acc_sh.at[idx_sl.at[1, j]], sem,
                       add=True)

    def _wait(sem, b):
      pltpu.make_async_copy(
          table_hbm.at[idx_sl.at[0, 0]], msgs_v.at[b], sem).wait()

    for sl in range(nslab):
      pltpu.sync_copy(src_hbm.at[wid, pl.ds(sl * sch, sch)], idx_sl.at[0])
      pltpu.sync_copy(dst_hbm.at[wid, pl.ds(sl * sch, sch)], idx_sl.at[1])
      for b in range(nbuf - 1):
        _gather(b, b, gsem[b])

      def _group(i, carry):
        for u in range(nbuf):
          _wait(gsem[u], u)
          _scatter(nbuf * i + u, u, ssem[u])
          if u == 0:
            @pl.when(i > 0)
            def _():
              _wait(ssem[nbuf - 1], nbuf - 1)
            _gather(nbuf * i + nbuf - 1, nbuf - 1, gsem[nbuf - 1])
          else:
            _wait(ssem[u - 1], u - 1)

            @pl.when(i < m - 1)
            def _():
              _gather(nbuf * i + u + nbuf - 1, u - 1, gsem[u - 1])
        return carry

      lax.fori_loop(0, m, _group, 0)
      _wait(ssem[nbuf - 1], nbuf - 1)
    plsc.subcore_barrier()

    rpt = _NPAD // _NS
    nout = rpt // ch
    for k in range(nout):
      b = k % nbuf
      if k >= nbuf:
        _wait(gsem[b], b)
      r0 = s * rpt + k * ch
      pltpu.sync_copy(acc_sh.at[pl.ds(r0, ch)], msgs_v.at[b])
      pltpu.async_copy(msgs_v.at[b], out_hbm.at[c, pl.ds(r0, ch)], gsem[b])
    for b in range(min(nbuf, nout)):
      _wait(gsem[b], b)

  return agg


_agg_x = _make_edge_agg(_F, 64, 40, 4)
_agg_q = _make_edge_agg(_CP, 128, 16, 4, tc_tiling=False)


def _tc_transform(p, w1, w2p):
  def body(p0, p1, a, b, o):
    t = jnp.dot(p0[...] + p1[...], a[...], preferred_element_type=jnp.float32)
    t = jnp.maximum(t, 0.0)
    o[...] = jnp.dot(t, b[...], preferred_element_type=jnp.float32)

  blk = 1280
  return pl.pallas_call(
      body,
      grid=(_NPAD // blk,),
      in_specs=[
          pl.BlockSpec((None, blk, _F), lambda i: (0, i, 0)),
          pl.BlockSpec((None, blk, _F), lambda i: (1, i, 0)),
          pl.BlockSpec((_F, _F), lambda i: (0, 0)),
          pl.BlockSpec((_F, _CP), lambda i: (0, 0)),
      ],
      out_specs=pl.BlockSpec((blk, _CP), lambda i: (i, 0)),
      out_shape=jax.ShapeDtypeStruct((_NPAD, _CP), jnp.float32),
  )(p, p, w1, w2p)


def _tc_pool(p2, batch2d):
  def body(p0, p1, b, o):
    h2 = p0[...] + p1[...]
    gids = lax.broadcasted_iota(jnp.int32, (_NPAD, _G), 1)
    onehot = jnp.where(b[...] == gids, 1.0, 0.0).astype(jnp.float32)
    sums = lax.dot_general(onehot, h2, (((0,), (0,)), ((), ())),
                           preferred_element_type=jnp.float32)
    counts = jnp.maximum(jnp.sum(onehot, axis=0), 1.0)
    o[...] = sums / counts[:, None]

  return pl.pallas_call(
      body,
      grid=(1,),
      in_specs=[
          pl.BlockSpec((None, _NPAD, _CP), lambda i: (0, 0, 0)),
          pl.BlockSpec((None, _NPAD, _CP), lambda i: (1, 0, 0)),
          pl.BlockSpec((_NPAD, 1), lambda i: (0, 0)),
      ],
      out_specs=pl.BlockSpec((_G, _CP), lambda i: (0, 0)),
      out_shape=jax.ShapeDtypeStruct((_G, _CP), jnp.float32),
  )(p2, p2, batch2d)


def kernel(x, edge_index, batch, W1, W2):
  src = edge_index[0].astype(jnp.int32)
  dst = edge_index[1].astype(jnp.int32)
  pad = _EPAD - _E
  ar = jnp.arange(pad, dtype=jnp.int32)
  src_p = jnp.concatenate([src, ar % _N])
  dst_p = jnp.concatenate([dst, _N + (ar % (_NPAD - _N))])

  p1 = _agg_x(x, src_p.reshape(_NW, _EPW // 64, 64),
              dst_p.reshape(_NW, _EPW // 64, 64))
  w2p = jnp.pad(W2, ((0, 0), (0, _CP - _C)))
  q = _tc_transform(p1, W1, w2p)
  p2 = _agg_q(q, src_p.reshape(_NW, _EPW // 128, 128),
              dst_p.reshape(_NW, _EPW // 128, 128))

  bpad = jnp.concatenate(
      [batch.astype(jnp.int32), jnp.full((_NPAD - _N,), _G, jnp.int32)])
  out = _tc_pool(p2, bpad.reshape(_NPAD, 1))
  return out[:, :_C]
